# Initial kernel scaffold; baseline (speedup 1.0000x reference)
import jax
import jax.numpy as jnp
from jax.experimental import pallas as pl


def kernel(x, edge_index, labels, lin1_w, lin1_b, c1_w, c1_u, c1_c, c1_b, c2_w, c2_u, c2_c, c2_b, c3_w, c3_u, c3_c, c3_b, lin2_w, lin2_b, lin3_w, lin3_b):
    raise NotImplementedError("write your pallas kernel here")


# speedup vs baseline: 10.8301x; 10.8301x over previous
import functools

import jax
import jax.numpy as jnp
from jax import lax
from jax.experimental import pallas as pl
from jax.experimental.pallas import tpu as pltpu
from jax.experimental.pallas import tpu_sc as plsc

N = 50000
NPAD = 50048
E = 800000
EPAD = 819200
ROWS = EPAD // 128
DUMP = 50040
TROWS = NPAD // 16
NBLK = 400
GRID_N = N // NBLK
RPC = 8

_SC_PARAMS = dict(compiler_params=pltpu.CompilerParams(use_tc_tiling_on_sc=False))


def _prep_body(src_ref, dst_ref, out_ref):
    s = src_ref[...]
    d = dst_ref[...]
    out_ref[...] = jnp.where(s == d, DUMP, d)


def _prep(srcp, dstp):
    return pl.pallas_call(
        _prep_body,
        grid=(ROWS // 64,),
        in_specs=[pl.BlockSpec((64, 128), lambda i: (i, 0)),
                  pl.BlockSpec((64, 128), lambda i: (i, 0))],
        out_specs=pl.BlockSpec((64, 128), lambda i: (i, 0)),
        out_shape=jax.ShapeDtypeStruct((ROWS, 128), jnp.int32),
    )(srcp, dstp)


def _lin1_body(x_ref, lab_ref, w_ref, b_ref, h_ref, pos_ref):
    i = pl.program_id(0)
    xb = x_ref[...]
    w = w_ref[...]
    h = (xb[:, 0:1] * w[0:1, :] + xb[:, 1:2] * w[1:2, :]
         + xb[:, 2:3] * w[2:3, :] + b_ref[...])
    h_ref[...] = jnp.maximum(h, 0.0)

    @pl.when(i == 0)
    def _():
        pos_ref[...] = jnp.zeros((1, 1), jnp.float32)

    pos_ref[...] += jnp.sum(lab_ref[...]).reshape(1, 1)


def _lin1(x, labels, w, b):
    return pl.pallas_call(
        _lin1_body,
        grid=(GRID_N,),
        in_specs=[pl.BlockSpec((NBLK, 3), lambda i: (i, 0)),
                  pl.BlockSpec((NBLK, 1), lambda i: (i, 0)),
                  pl.BlockSpec((3, 16), lambda i: (0, 0)),
                  pl.BlockSpec((1, 16), lambda i: (0, 0))],
        out_specs=[pl.BlockSpec((NBLK, 16), lambda i: (i, 0)),
                   pl.BlockSpec((1, 1), lambda i: (0, 0))],
        out_shape=[jax.ShapeDtypeStruct((N, 16), jnp.float32),
                   jax.ShapeDtypeStruct((1, 1), jnp.float32)],
    )(x, labels, w, b)


def _zero_own(acc, zb, wlo):
    pltpu.sync_copy(zb, acc.at[pl.ds(wlo, TROWS)])


def _scatter_pass(table, srcp, dstm, idx_s, idx_d, rows, acc, sem, base, nch):
    def chunk(ch, _):
        lo = base + ch * RPC
        pltpu.sync_copy(srcp.at[pl.ds(lo, RPC)], idx_s)
        pltpu.sync_copy(dstm.at[pl.ds(lo, RPC)], idx_d)
        cps = [pltpu.async_copy(table.at[idx_s.at[j]],
                                rows.at[pl.ds(j * 128, 128)], sem)
               for j in range(RPC)]
        for cp in cps:
            cp.wait()
        for j in range(RPC):
            pltpu.sync_copy(rows.at[pl.ds(j * 128, 128)],
                            acc.at[idx_d.at[j]], add=True)
        return 0

    lax.fori_loop(0, nch, chunk, 0)


def _count_pass(dstm, idx_d, ones_b, acc, base, nch):
    def chunk(ch, _):
        lo = base + ch * RPC
        pltpu.sync_copy(dstm.at[pl.ds(lo, RPC)], idx_d)
        for j in range(RPC):
            pltpu.sync_copy(ones_b, acc.at[idx_d.at[j]], add=True)
        return 0

    lax.fori_loop(0, nch, chunk, 0)


def _writeout(acc, out, wlo):
    pltpu.sync_copy(acc.at[pl.ds(wlo, TROWS)], out.at[pl.ds(wlo, TROWS)])


def _make_conv1(mesh):
    @functools.partial(
        pl.kernel,
        out_type=(jax.ShapeDtypeStruct((NPAD, 16), jnp.float32),
                  jax.ShapeDtypeStruct((NPAD, 16), jnp.float32),
                  jax.ShapeDtypeStruct((NPAD, 16), jnp.float32),
                  jax.ShapeDtypeStruct((NPAD, 16), jnp.float32)),
        mesh=mesh,
        scratch_types=[
            pltpu.VMEM((RPC, 128), jnp.int32),
            pltpu.VMEM((RPC, 128), jnp.int32),
            pltpu.VMEM((RPC * 128, 16), jnp.float32),
            pltpu.VMEM((128, 16), jnp.float32),
            pltpu.VMEM_SHARED((NPAD, 16), jnp.float32),
            pltpu.SemaphoreType.DMA,
        ],
        **_SC_PARAMS,
    )
    def _conv1(h1, srcp, dstm, zb, ones_h, s1_0, s1_1, ct_0, ct_1,
               idx_s, idx_d, rows, ones_b, acc, sem):
        c = lax.axis_index("c")
        s = lax.axis_index("s")
        wlo = s * TROWS
        base = (c * 16 + s) * (ROWS // 32)
        nch = (ROWS // 32) // RPC

        pltpu.sync_copy(ones_h, ones_b)
        _zero_own(acc, zb, wlo)
        plsc.subcore_barrier()
        _scatter_pass(h1, srcp, dstm, idx_s, idx_d, rows, acc, sem, base, nch)
        plsc.subcore_barrier()

        @pl.when(c == 0)
        def _():
            _writeout(acc, s1_0, wlo)

        @pl.when(c == 1)
        def _():
            _writeout(acc, s1_1, wlo)

        _zero_own(acc, zb, wlo)
        plsc.subcore_barrier()
        _count_pass(dstm, idx_d, ones_b, acc, base, nch)
        plsc.subcore_barrier()

        @pl.when(c == 0)
        def _():
            _writeout(acc, ct_0, wlo)

        @pl.when(c == 1)
        def _():
            _writeout(acc, ct_1, wlo)

    return _conv1


def _make_conv2(mesh):
    @functools.partial(
        pl.kernel,
        out_type=(jax.ShapeDtypeStruct((NPAD, 16), jnp.float32),
                  jax.ShapeDtypeStruct((NPAD, 16), jnp.float32)),
        mesh=mesh,
        scratch_types=[
            pltpu.VMEM((RPC, 128), jnp.int32),
            pltpu.VMEM((RPC, 128), jnp.int32),
            pltpu.VMEM((RPC * 128, 16), jnp.float32),
            pltpu.VMEM_SHARED((NPAD, 16), jnp.float32),
            pltpu.SemaphoreType.DMA,
        ],
        **_SC_PARAMS,
    )
    def _conv2(h2a, h2b, srcp, dstm, zb, s2a, s2b,
               idx_s, idx_d, rows, acc, sem):
        c = lax.axis_index("c")
        s = lax.axis_index("s")
        wlo = s * TROWS
        base = s * (ROWS // 16)
        nch = (ROWS // 16) // RPC

        _zero_own(acc, zb, wlo)
        plsc.subcore_barrier()

        @pl.when(c == 0)
        def _():
            _scatter_pass(h2a, srcp, dstm, idx_s, idx_d, rows, acc, sem,
                          base, nch)
            plsc.subcore_barrier()
            _writeout(acc, s2a, wlo)

        @pl.when(c == 1)
        def _():
            _scatter_pass(h2b, srcp, dstm, idx_s, idx_d, rows, acc, sem,
                          base, nch)
            plsc.subcore_barrier()
            _writeout(acc, s2b, wlo)

    return _conv2


def _make_conv3(mesh):
    @functools.partial(
        pl.kernel,
        out_type=(jax.ShapeDtypeStruct((NPAD, 16), jnp.float32),
                  jax.ShapeDtypeStruct((NPAD, 16), jnp.float32),
                  jax.ShapeDtypeStruct((NPAD, 16), jnp.float32),
                  jax.ShapeDtypeStruct((NPAD, 16), jnp.float32)),
        mesh=mesh,
        scratch_types=[
            pltpu.VMEM((RPC, 128), jnp.int32),
            pltpu.VMEM((RPC, 128), jnp.int32),
            pltpu.VMEM((RPC * 128, 16), jnp.float32),
            pltpu.VMEM_SHARED((NPAD, 16), jnp.float32),
            pltpu.SemaphoreType.DMA,
        ],
        **_SC_PARAMS,
    )
    def _conv3(h3a, h3b, h3c, h3d, srcp, dstm, zb, s3a, s3b, s3c, s3d,
               idx_s, idx_d, rows, acc, sem):
        c = lax.axis_index("c")
        s = lax.axis_index("s")
        wlo = s * TROWS
        base = s * (ROWS // 16)
        nch = (ROWS // 16) // RPC

        def run(t0, t1, o0, o1):
            _zero_own(acc, zb, wlo)
            plsc.subcore_barrier()
            _scatter_pass(t0, srcp, dstm, idx_s, idx_d, rows, acc, sem,
                          base, nch)
            plsc.subcore_barrier()
            _writeout(acc, o0, wlo)
            _zero_own(acc, zb, wlo)
            plsc.subcore_barrier()
            _scatter_pass(t1, srcp, dstm, idx_s, idx_d, rows, acc, sem,
                          base, nch)
            plsc.subcore_barrier()
            _writeout(acc, o1, wlo)

        @pl.when(c == 0)
        def _():
            run(h3a, h3b, s3a, s3b)

        @pl.when(c == 1)
        def _():
            run(h3c, h3d, s3c, s3d)

    return _conv3


@functools.lru_cache(maxsize=None)
def _sc_kernels():
    mesh = plsc.VectorSubcoreMesh(core_axis_name="c", subcore_axis_name="s")
    return (_make_conv1(mesh), _make_conv2(mesh), _make_conv3(mesh))


def _comb1_body(s0_ref, s1_ref, c0_ref, c1_ref, h1_ref, w_ref, b_ref,
                ha_ref, hb_ref, rc_ref):
    cnt = c0_ref[...] + c1_ref[...] + 1.0
    g = (s0_ref[...] + s1_ref[...] + h1_ref[...]) / cnt
    h2 = jnp.dot(g, w_ref[...], preferred_element_type=jnp.float32) + b_ref[...]
    h2 = jnp.maximum(h2, 0.0)
    ha_ref[...] = h2[:, 0:16]
    hb_ref[...] = h2[:, 16:32]
    rc_ref[...] = 1.0 / cnt


def _comb1(s0, s1, c0, c1, h1, w, b):
    return pl.pallas_call(
        _comb1_body,
        grid=(GRID_N,),
        in_specs=[pl.BlockSpec((NBLK, 16), lambda i: (i, 0)),
                  pl.BlockSpec((NBLK, 16), lambda i: (i, 0)),
                  pl.BlockSpec((NBLK, 16), lambda i: (i, 0)),
                  pl.BlockSpec((NBLK, 16), lambda i: (i, 0)),
                  pl.BlockSpec((NBLK, 16), lambda i: (i, 0)),
                  pl.BlockSpec((16, 32), lambda i: (0, 0)),
                  pl.BlockSpec((1, 32), lambda i: (0, 0))],
        out_specs=[pl.BlockSpec((NBLK, 16), lambda i: (i, 0)),
                   pl.BlockSpec((NBLK, 16), lambda i: (i, 0)),
                   pl.BlockSpec((NBLK, 16), lambda i: (i, 0))],
        out_shape=[jax.ShapeDtypeStruct((N, 16), jnp.float32),
                   jax.ShapeDtypeStruct((N, 16), jnp.float32),
                   jax.ShapeDtypeStruct((N, 16), jnp.float32)],
    )(s0, s1, c0, c1, h1, w, b)


def _comb2_body(sa_ref, sb_ref, ha_ref, hb_ref, rc_ref, w_ref, b_ref,
                oa_ref, ob_ref, oc_ref, od_ref):
    rc = rc_ref[...]
    rc32 = jnp.concatenate([rc, rc], axis=1)
    g = jnp.concatenate([sa_ref[...] + ha_ref[...],
                         sb_ref[...] + hb_ref[...]], axis=1) * rc32
    h3 = jnp.dot(g, w_ref[...], preferred_element_type=jnp.float32) + b_ref[...]
    h3 = jnp.maximum(h3, 0.0)
    oa_ref[...] = h3[:, 0:16]
    ob_ref[...] = h3[:, 16:32]
    oc_ref[...] = h3[:, 32:48]
    od_ref[...] = h3[:, 48:64]


def _comb2(sa, sb, ha, hb, rc, w, b):
    return pl.pallas_call(
        _comb2_body,
        grid=(GRID_N,),
        in_specs=[pl.BlockSpec((NBLK, 16), lambda i: (i, 0)),
                  pl.BlockSpec((NBLK, 16), lambda i: (i, 0)),
                  pl.BlockSpec((NBLK, 16), lambda i: (i, 0)),
                  pl.BlockSpec((NBLK, 16), lambda i: (i, 0)),
                  pl.BlockSpec((NBLK, 16), lambda i: (i, 0)),
                  pl.BlockSpec((32, 64), lambda i: (0, 0)),
                  pl.BlockSpec((1, 64), lambda i: (0, 0))],
        out_specs=[pl.BlockSpec((NBLK, 16), lambda i: (i, 0)),
                   pl.BlockSpec((NBLK, 16), lambda i: (i, 0)),
                   pl.BlockSpec((NBLK, 16), lambda i: (i, 0)),
                   pl.BlockSpec((NBLK, 16), lambda i: (i, 0))],
        out_shape=[jax.ShapeDtypeStruct((N, 16), jnp.float32)] * 4,
    )(sa, sb, ha, hb, rc, w, b)


def _final_body(sa_ref, sb_ref, sc_ref, sd_ref,
                ha_ref, hb_ref, hc_ref, hd_ref, rc_ref, lab_ref, pos_ref,
                c3w_ref, c3b_ref, w2_ref, b2_ref, w3_ref, b3_ref,
                p_ref, loss_ref, acc_ref):
    i = pl.program_id(0)
    rc = rc_ref[...]
    rc64 = jnp.concatenate([rc, rc, rc, rc], axis=1)
    g = jnp.concatenate([sa_ref[...] + ha_ref[...],
                         sb_ref[...] + hb_ref[...],
                         sc_ref[...] + hc_ref[...],
                         sd_ref[...] + hd_ref[...]], axis=1) * rc64
    h4 = jnp.dot(g, c3w_ref[...], preferred_element_type=jnp.float32) + c3b_ref[...]
    h4 = jnp.maximum(h4, 0.0)
    h5 = jnp.dot(h4, w2_ref[...], preferred_element_type=jnp.float32) + b2_ref[...]
    h5 = jnp.maximum(h5, 0.0)
    z = jnp.dot(h5, w3_ref[...], preferred_element_type=jnp.float32) + b3_ref[...]
    p = jnp.clip(jax.nn.sigmoid(z), 1e-12, 1.0 - 1e-7)
    p_ref[...] = p

    lb = lab_ref[...]
    posv = pos_ref[0]
    negv = float(N) - posv
    wpos = float(N) / (2.0 * jnp.maximum(posv, 1.0))
    wneg = float(N) / (2.0 * jnp.maximum(negv, 1.0))
    wgt = jnp.where(lb > 0.5, wpos, wneg)
    ll = (lb * jnp.maximum(jnp.log(p), -100.0)
          + (1.0 - lb) * jnp.maximum(jnp.log(1.0 - p), -100.0))

    @pl.when(i == 0)
    def _():
        acc_ref[0] = 0.0

    acc_ref[0] += jnp.sum(-wgt * ll)

    @pl.when(i == GRID_N - 1)
    def _():
        loss_ref[...] = (acc_ref[0] / float(N)).reshape(1, 1)


def _final(s3, h3, rc, labels, pos, c3w, c3b, w2, b2, w3, b3):
    blk16 = pl.BlockSpec((NBLK, 16), lambda i: (i, 0))
    return pl.pallas_call(
        _final_body,
        grid=(GRID_N,),
        in_specs=[blk16, blk16, blk16, blk16,
                  blk16, blk16, blk16, blk16,
                  blk16,
                  pl.BlockSpec((NBLK, 1), lambda i: (i, 0)),
                  pl.BlockSpec(memory_space=pltpu.SMEM),
                  pl.BlockSpec((64, 128), lambda i: (0, 0)),
                  pl.BlockSpec((1, 128), lambda i: (0, 0)),
                  pl.BlockSpec((128, 1024), lambda i: (0, 0)),
                  pl.BlockSpec((1, 1024), lambda i: (0, 0)),
                  pl.BlockSpec((1024, 1), lambda i: (0, 0)),
                  pl.BlockSpec((1, 1), lambda i: (0, 0))],
        out_specs=[pl.BlockSpec((NBLK, 1), lambda i: (i, 0)),
                   pl.BlockSpec((1, 1), lambda i: (0, 0))],
        out_shape=[jax.ShapeDtypeStruct((N, 1), jnp.float32),
                   jax.ShapeDtypeStruct((1, 1), jnp.float32)],
        scratch_shapes=[pltpu.SMEM((1,), jnp.float32)],
    )(*s3, *h3, rc, labels, pos, c3w, c3b, w2, b2, w3, b3)


def kernel(x, edge_index, labels, lin1_w, lin1_b, c1_w, c1_u, c1_c, c1_b,
           c2_w, c2_u, c2_c, c2_b, c3_w, c3_u, c3_c, c3_b,
           lin2_w, lin2_b, lin3_w, lin3_b):
    conv1, conv2, conv3 = _sc_kernels()
    src = edge_index[0]
    dst = edge_index[1]
    pad = jnp.zeros((EPAD - E,), jnp.int32)
    srcp = jnp.concatenate([src, pad]).reshape(ROWS, 128)
    dstp = jnp.concatenate([dst, pad]).reshape(ROWS, 128)

    dstm = _prep(srcp, dstp)
    h1, pos = _lin1(x, labels, lin1_w, lin1_b.reshape(1, 16))

    zb16 = jnp.zeros((TROWS, 16), jnp.float32)
    ones128 = jnp.ones((128, 16), jnp.float32)

    s1_0, s1_1, ct_0, ct_1 = conv1(h1, srcp, dstm, zb16, ones128)
    h2a, h2b, rc = _comb1(s1_0, s1_1, ct_0, ct_1, h1, c1_w,
                          c1_b.reshape(1, 32))
    s2a, s2b = conv2(h2a, h2b, srcp, dstm, zb16)
    h3a, h3b, h3c, h3d = _comb2(s2a, s2b, h2a, h2b, rc, c2_w,
                                c2_b.reshape(1, 64))
    s3 = conv3(h3a, h3b, h3c, h3d, srcp, dstm, zb16)
    p, loss = _final(s3, (h3a, h3b, h3c, h3d), rc, labels, pos.reshape(1),
                     c3_w, c3_b.reshape(1, 128), lin2_w,
                     lin2_b.reshape(1, 1024), lin3_w, lin3_b.reshape(1, 1))
    return loss[0, 0], p


# speedup vs baseline: 14.6259x; 1.3505x over previous
import functools

import jax
import jax.numpy as jnp
from jax import lax
from jax.experimental import pallas as pl
from jax.experimental.pallas import tpu as pltpu
from jax.experimental.pallas import tpu_sc as plsc

N = 50000
NPAD = 50048
E = 800000
EPAD = 819200
ROWS = EPAD // 128
DUMP = 50040
TROWS = NPAD // 16
NBLK = 2000
GRID_N = N // NBLK
RPC = 10

_SC_PARAMS = dict(compiler_params=pltpu.CompilerParams(use_tc_tiling_on_sc=False))


def _prep_body(src_ref, dst_ref, out_ref):
    s = src_ref[...]
    d = dst_ref[...]
    out_ref[...] = jnp.where(s == d, DUMP, d)


def _prep(srcp, dstp):
    return pl.pallas_call(
        _prep_body,
        grid=(ROWS // 64,),
        in_specs=[pl.BlockSpec((64, 128), lambda i: (i, 0)),
                  pl.BlockSpec((64, 128), lambda i: (i, 0))],
        out_specs=pl.BlockSpec((64, 128), lambda i: (i, 0)),
        out_shape=jax.ShapeDtypeStruct((ROWS, 128), jnp.int32),
    )(srcp, dstp)


def _lin1_body(x_ref, lab_ref, w_ref, b_ref, h_ref, pos_ref):
    i = pl.program_id(0)
    xb = x_ref[...]
    w = w_ref[...]
    h = (xb[:, 0:1] * w[0:1, :] + xb[:, 1:2] * w[1:2, :]
         + xb[:, 2:3] * w[2:3, :] + b_ref[...])
    h_ref[...] = jnp.maximum(h, 0.0)

    @pl.when(i == 0)
    def _():
        pos_ref[...] = jnp.zeros((1, 1), jnp.float32)

    pos_ref[...] += jnp.sum(lab_ref[...]).reshape(1, 1)


def _lin1(x, labels, w, b):
    return pl.pallas_call(
        _lin1_body,
        grid=(GRID_N,),
        in_specs=[pl.BlockSpec((NBLK, 3), lambda i: (i, 0)),
                  pl.BlockSpec((NBLK, 1), lambda i: (i, 0)),
                  pl.BlockSpec((3, 16), lambda i: (0, 0)),
                  pl.BlockSpec((1, 16), lambda i: (0, 0))],
        out_specs=[pl.BlockSpec((NBLK, 16), lambda i: (i, 0)),
                   pl.BlockSpec((1, 1), lambda i: (0, 0))],
        out_shape=[jax.ShapeDtypeStruct((N, 16), jnp.float32),
                   jax.ShapeDtypeStruct((1, 1), jnp.float32)],
    )(x, labels, w, b)


def _zero_own(acc, zb, wlo):
    pltpu.sync_copy(zb, acc.at[pl.ds(wlo, TROWS)])


def _load_idx(srcp, dstm, lo, i_ref, d_ref):
    pltpu.sync_copy(srcp.at[pl.ds(lo, RPC)], i_ref)
    pltpu.sync_copy(dstm.at[pl.ds(lo, RPC)], d_ref)


def _fire_g(table, i_ref, r_ref, sem):
    for j in range(RPC):
        pltpu.async_copy(table.at[i_ref.at[j]],
                         r_ref.at[pl.ds(j * 128, 128)], sem)


def _wait_g(table, i_ref, r_ref, sem):
    for j in range(RPC):
        pltpu.make_async_copy(table.at[i_ref.at[j]],
                              r_ref.at[pl.ds(j * 128, 128)], sem).wait()


def _fire_s(acc, d_ref, r_ref, sem):
    for j in range(RPC):
        pltpu.async_copy(r_ref.at[pl.ds(j * 128, 128)],
                         acc.at[d_ref.at[j]], sem, add=True)


def _wait_s(acc, d_ref, r_ref, sem):
    for j in range(RPC):
        pltpu.make_async_copy(r_ref.at[pl.ds(j * 128, 128)],
                              acc.at[d_ref.at[j]], sem).wait()


def _scatter_pass(table, srcp, dstm, bufs, acc, base, nch):
    iA, dA, rA, sgA, ssA, iB, dB, rB, sgB, ssB = bufs

    _load_idx(srcp, dstm, base, iA, dA)
    _fire_g(table, iA, rA, sgA)
    _load_idx(srcp, dstm, base + RPC, iB, dB)
    _fire_g(table, iB, rB, sgB)

    def body(k, _):
        _wait_g(table, iA, rA, sgA)
        _fire_s(acc, dA, rA, ssA)
        _wait_g(table, iB, rB, sgB)
        _fire_s(acc, dB, rB, ssB)

        @pl.when(k < nch // 2 - 1)
        def _():
            _wait_s(acc, dA, rA, ssA)
            _load_idx(srcp, dstm, base + (2 * k + 2) * RPC, iA, dA)
            _fire_g(table, iA, rA, sgA)
            _wait_s(acc, dB, rB, ssB)
            _load_idx(srcp, dstm, base + (2 * k + 3) * RPC, iB, dB)
            _fire_g(table, iB, rB, sgB)

        return 0

    lax.fori_loop(0, nch // 2, body, 0)
    _wait_s(acc, dA, rA, ssA)
    _wait_s(acc, dB, rB, ssB)


def _count_pass(srcp, dstm, bufs, ones_b, acc, base, nch):
    iA, dA, rA, sgA, ssA, iB, dB, rB, sgB, ssB = bufs

    _load_idx(srcp, dstm, base, iA, dA)
    _load_idx(srcp, dstm, base + RPC, iB, dB)

    def fire(d_ref, sem):
        for j in range(RPC):
            pltpu.async_copy(ones_b, acc.at[d_ref.at[j]], sem, add=True)

    def drain(d_ref, sem):
        for j in range(RPC):
            pltpu.make_async_copy(ones_b, acc.at[d_ref.at[j]], sem).wait()

    def body(k, _):
        fire(dA, ssA)
        fire(dB, ssB)

        @pl.when(k < nch // 2 - 1)
        def _():
            drain(dA, ssA)
            _load_idx(srcp, dstm, base + (2 * k + 2) * RPC, iA, dA)
            drain(dB, ssB)
            _load_idx(srcp, dstm, base + (2 * k + 3) * RPC, iB, dB)

        return 0

    lax.fori_loop(0, nch // 2, body, 0)
    drain(dA, ssA)
    drain(dB, ssB)


def _writeout(acc, out, wlo):
    pltpu.sync_copy(acc.at[pl.ds(wlo, TROWS)], out.at[pl.ds(wlo, TROWS)])


_CONV_SCRATCH = [
    pltpu.VMEM((RPC, 128), jnp.int32),
    pltpu.VMEM((RPC, 128), jnp.int32),
    pltpu.VMEM((RPC * 128, 16), jnp.float32),
    pltpu.SemaphoreType.DMA,
    pltpu.SemaphoreType.DMA,
    pltpu.VMEM((RPC, 128), jnp.int32),
    pltpu.VMEM((RPC, 128), jnp.int32),
    pltpu.VMEM((RPC * 128, 16), jnp.float32),
    pltpu.SemaphoreType.DMA,
    pltpu.SemaphoreType.DMA,
]


def _make_conv1(mesh):
    @functools.partial(
        pl.kernel,
        out_type=(jax.ShapeDtypeStruct((NPAD, 16), jnp.float32),
                  jax.ShapeDtypeStruct((NPAD, 16), jnp.float32),
                  jax.ShapeDtypeStruct((NPAD, 16), jnp.float32),
                  jax.ShapeDtypeStruct((NPAD, 16), jnp.float32)),
        mesh=mesh,
        scratch_types=_CONV_SCRATCH + [
            pltpu.VMEM((128, 16), jnp.float32),
            pltpu.VMEM_SHARED((NPAD, 16), jnp.float32),
        ],
        **_SC_PARAMS,
    )
    def _conv1(h1, srcp, dstm, zb, ones_h, s1_0, s1_1, ct_0, ct_1,
               *scratch):
        bufs = scratch[:10]
        ones_b, acc = scratch[10], scratch[11]
        c = lax.axis_index("c")
        s = lax.axis_index("s")
        wlo = s * TROWS
        base = (c * 16 + s) * (ROWS // 32)
        nch = (ROWS // 32) // RPC

        pltpu.sync_copy(ones_h, ones_b)
        _zero_own(acc, zb, wlo)
        plsc.subcore_barrier()
        _scatter_pass(h1, srcp, dstm, bufs, acc, base, nch)
        plsc.subcore_barrier()

        @pl.when(c == 0)
        def _():
            _writeout(acc, s1_0, wlo)

        @pl.when(c == 1)
        def _():
            _writeout(acc, s1_1, wlo)

        _zero_own(acc, zb, wlo)
        plsc.subcore_barrier()
        _count_pass(srcp, dstm, bufs, ones_b, acc, base, nch)
        plsc.subcore_barrier()

        @pl.when(c == 0)
        def _():
            _writeout(acc, ct_0, wlo)

        @pl.when(c == 1)
        def _():
            _writeout(acc, ct_1, wlo)

    return _conv1


def _make_conv2(mesh):
    @functools.partial(
        pl.kernel,
        out_type=(jax.ShapeDtypeStruct((NPAD, 16), jnp.float32),
                  jax.ShapeDtypeStruct((NPAD, 16), jnp.float32)),
        mesh=mesh,
        scratch_types=_CONV_SCRATCH + [
            pltpu.VMEM_SHARED((NPAD, 16), jnp.float32),
        ],
        **_SC_PARAMS,
    )
    def _conv2(h2a, h2b, srcp, dstm, zb, s2a, s2b, *scratch):
        bufs = scratch[:10]
        acc = scratch[10]
        c = lax.axis_index("c")
        s = lax.axis_index("s")
        wlo = s * TROWS
        base = s * (ROWS // 16)
        nch = (ROWS // 16) // RPC

        _zero_own(acc, zb, wlo)
        plsc.subcore_barrier()

        @pl.when(c == 0)
        def _():
            _scatter_pass(h2a, srcp, dstm, bufs, acc, base, nch)
            plsc.subcore_barrier()
            _writeout(acc, s2a, wlo)

        @pl.when(c == 1)
        def _():
            _scatter_pass(h2b, srcp, dstm, bufs, acc, base, nch)
            plsc.subcore_barrier()
            _writeout(acc, s2b, wlo)

    return _conv2


def _make_conv3(mesh):
    @functools.partial(
        pl.kernel,
        out_type=(jax.ShapeDtypeStruct((NPAD, 16), jnp.float32),
                  jax.ShapeDtypeStruct((NPAD, 16), jnp.float32),
                  jax.ShapeDtypeStruct((NPAD, 16), jnp.float32),
                  jax.ShapeDtypeStruct((NPAD, 16), jnp.float32)),
        mesh=mesh,
        scratch_types=_CONV_SCRATCH + [
            pltpu.VMEM_SHARED((NPAD, 16), jnp.float32),
        ],
        **_SC_PARAMS,
    )
    def _conv3(h3a, h3b, h3c, h3d, srcp, dstm, zb, s3a, s3b, s3c, s3d,
               *scratch):
        bufs = scratch[:10]
        acc = scratch[10]
        c = lax.axis_index("c")
        s = lax.axis_index("s")
        wlo = s * TROWS
        base = s * (ROWS // 16)
        nch = (ROWS // 16) // RPC

        def run(t0, t1, o0, o1):
            _zero_own(acc, zb, wlo)
            plsc.subcore_barrier()
            _scatter_pass(t0, srcp, dstm, bufs, acc, base, nch)
            plsc.subcore_barrier()
            _writeout(acc, o0, wlo)
            _zero_own(acc, zb, wlo)
            plsc.subcore_barrier()
            _scatter_pass(t1, srcp, dstm, bufs, acc, base, nch)
            plsc.subcore_barrier()
            _writeout(acc, o1, wlo)

        @pl.when(c == 0)
        def _():
            run(h3a, h3b, s3a, s3b)

        @pl.when(c == 1)
        def _():
            run(h3c, h3d, s3c, s3d)

    return _conv3


@functools.lru_cache(maxsize=None)
def _sc_kernels():
    mesh = plsc.VectorSubcoreMesh(core_axis_name="c", subcore_axis_name="s")
    return (_make_conv1(mesh), _make_conv2(mesh), _make_conv3(mesh))


def _comb1_body(s0_ref, s1_ref, c0_ref, c1_ref, h1_ref, w_ref, b_ref,
                ha_ref, hb_ref, rc_ref):
    cnt = c0_ref[...] + c1_ref[...] + 1.0
    g = (s0_ref[...] + s1_ref[...] + h1_ref[...]) / cnt
    h2 = jnp.dot(g, w_ref[...], preferred_element_type=jnp.float32) + b_ref[...]
    h2 = jnp.maximum(h2, 0.0)
    ha_ref[...] = h2[:, 0:16]
    hb_ref[...] = h2[:, 16:32]
    rc_ref[...] = 1.0 / cnt


def _comb1(s0, s1, c0, c1, h1, w, b):
    return pl.pallas_call(
        _comb1_body,
        grid=(GRID_N,),
        in_specs=[pl.BlockSpec((NBLK, 16), lambda i: (i, 0)),
                  pl.BlockSpec((NBLK, 16), lambda i: (i, 0)),
                  pl.BlockSpec((NBLK, 16), lambda i: (i, 0)),
                  pl.BlockSpec((NBLK, 16), lambda i: (i, 0)),
                  pl.BlockSpec((NBLK, 16), lambda i: (i, 0)),
                  pl.BlockSpec((16, 32), lambda i: (0, 0)),
                  pl.BlockSpec((1, 32), lambda i: (0, 0))],
        out_specs=[pl.BlockSpec((NBLK, 16), lambda i: (i, 0)),
                   pl.BlockSpec((NBLK, 16), lambda i: (i, 0)),
                   pl.BlockSpec((NBLK, 16), lambda i: (i, 0))],
        out_shape=[jax.ShapeDtypeStruct((N, 16), jnp.float32),
                   jax.ShapeDtypeStruct((N, 16), jnp.float32),
                   jax.ShapeDtypeStruct((N, 16), jnp.float32)],
    )(s0, s1, c0, c1, h1, w, b)


def _comb2_body(sa_ref, sb_ref, ha_ref, hb_ref, rc_ref, w_ref, b_ref,
                oa_ref, ob_ref, oc_ref, od_ref):
    rc = rc_ref[...]
    rc32 = jnp.concatenate([rc, rc], axis=1)
    g = jnp.concatenate([sa_ref[...] + ha_ref[...],
                         sb_ref[...] + hb_ref[...]], axis=1) * rc32
    h3 = jnp.dot(g, w_ref[...], preferred_element_type=jnp.float32) + b_ref[...]
    h3 = jnp.maximum(h3, 0.0)
    oa_ref[...] = h3[:, 0:16]
    ob_ref[...] = h3[:, 16:32]
    oc_ref[...] = h3[:, 32:48]
    od_ref[...] = h3[:, 48:64]


def _comb2(sa, sb, ha, hb, rc, w, b):
    return pl.pallas_call(
        _comb2_body,
        grid=(GRID_N,),
        in_specs=[pl.BlockSpec((NBLK, 16), lambda i: (i, 0)),
                  pl.BlockSpec((NBLK, 16), lambda i: (i, 0)),
                  pl.BlockSpec((NBLK, 16), lambda i: (i, 0)),
                  pl.BlockSpec((NBLK, 16), lambda i: (i, 0)),
                  pl.BlockSpec((NBLK, 16), lambda i: (i, 0)),
                  pl.BlockSpec((32, 64), lambda i: (0, 0)),
                  pl.BlockSpec((1, 64), lambda i: (0, 0))],
        out_specs=[pl.BlockSpec((NBLK, 16), lambda i: (i, 0)),
                   pl.BlockSpec((NBLK, 16), lambda i: (i, 0)),
                   pl.BlockSpec((NBLK, 16), lambda i: (i, 0)),
                   pl.BlockSpec((NBLK, 16), lambda i: (i, 0))],
        out_shape=[jax.ShapeDtypeStruct((N, 16), jnp.float32)] * 4,
    )(sa, sb, ha, hb, rc, w, b)


def _final_body(sa_ref, sb_ref, sc_ref, sd_ref,
                ha_ref, hb_ref, hc_ref, hd_ref, rc_ref, lab_ref, pos_ref,
                c3w_ref, c3b_ref, w2_ref, b2_ref, w3_ref, b3_ref,
                p_ref, loss_ref, acc_ref):
    i = pl.program_id(0)
    rc = rc_ref[...]
    rc64 = jnp.concatenate([rc, rc, rc, rc], axis=1)
    g = jnp.concatenate([sa_ref[...] + ha_ref[...],
                         sb_ref[...] + hb_ref[...],
                         sc_ref[...] + hc_ref[...],
                         sd_ref[...] + hd_ref[...]], axis=1) * rc64
    h4 = jnp.dot(g, c3w_ref[...], preferred_element_type=jnp.float32) + c3b_ref[...]
    h4 = jnp.maximum(h4, 0.0)
    h5 = jnp.dot(h4, w2_ref[...], preferred_element_type=jnp.float32) + b2_ref[...]
    h5 = jnp.maximum(h5, 0.0)
    z = jnp.dot(h5, w3_ref[...], preferred_element_type=jnp.float32) + b3_ref[...]
    p = jnp.clip(jax.nn.sigmoid(z), 1e-12, 1.0 - 1e-7)
    p_ref[...] = p

    lb = lab_ref[...]
    posv = pos_ref[0]
    negv = float(N) - posv
    wpos = float(N) / (2.0 * jnp.maximum(posv, 1.0))
    wneg = float(N) / (2.0 * jnp.maximum(negv, 1.0))
    wgt = jnp.where(lb > 0.5, wpos, wneg)
    ll = (lb * jnp.maximum(jnp.log(p), -100.0)
          + (1.0 - lb) * jnp.maximum(jnp.log(1.0 - p), -100.0))

    @pl.when(i == 0)
    def _():
        acc_ref[0] = 0.0

    acc_ref[0] += jnp.sum(-wgt * ll)

    @pl.when(i == GRID_N - 1)
    def _():
        loss_ref[...] = (acc_ref[0] / float(N)).reshape(1, 1)


def _final(s3, h3, rc, labels, pos, c3w, c3b, w2, b2, w3, b3):
    blk16 = pl.BlockSpec((NBLK, 16), lambda i: (i, 0))
    return pl.pallas_call(
        _final_body,
        grid=(GRID_N,),
        in_specs=[blk16, blk16, blk16, blk16,
                  blk16, blk16, blk16, blk16,
                  blk16,
                  pl.BlockSpec((NBLK, 1), lambda i: (i, 0)),
                  pl.BlockSpec(memory_space=pltpu.SMEM),
                  pl.BlockSpec((64, 128), lambda i: (0, 0)),
                  pl.BlockSpec((1, 128), lambda i: (0, 0)),
                  pl.BlockSpec((128, 1024), lambda i: (0, 0)),
                  pl.BlockSpec((1, 1024), lambda i: (0, 0)),
                  pl.BlockSpec((1024, 1), lambda i: (0, 0)),
                  pl.BlockSpec((1, 1), lambda i: (0, 0))],
        out_specs=[pl.BlockSpec((NBLK, 1), lambda i: (i, 0)),
                   pl.BlockSpec((1, 1), lambda i: (0, 0))],
        out_shape=[jax.ShapeDtypeStruct((N, 1), jnp.float32),
                   jax.ShapeDtypeStruct((1, 1), jnp.float32)],
        scratch_shapes=[pltpu.SMEM((1,), jnp.float32)],
    )(*s3, *h3, rc, labels, pos, c3w, c3b, w2, b2, w3, b3)


def kernel(x, edge_index, labels, lin1_w, lin1_b, c1_w, c1_u, c1_c, c1_b,
           c2_w, c2_u, c2_c, c2_b, c3_w, c3_u, c3_c, c3_b,
           lin2_w, lin2_b, lin3_w, lin3_b):
    conv1, conv2, conv3 = _sc_kernels()
    src = edge_index[0]
    dst = edge_index[1]
    pad = jnp.zeros((EPAD - E,), jnp.int32)
    srcp = jnp.concatenate([src, pad]).reshape(ROWS, 128)
    dstp = jnp.concatenate([dst, pad]).reshape(ROWS, 128)

    dstm = _prep(srcp, dstp)
    h1, pos = _lin1(x, labels, lin1_w, lin1_b.reshape(1, 16))

    zb16 = jnp.zeros((TROWS, 16), jnp.float32)
    ones128 = jnp.ones((128, 16), jnp.float32)

    s1_0, s1_1, ct_0, ct_1 = conv1(h1, srcp, dstm, zb16, ones128)
    h2a, h2b, rc = _comb1(s1_0, s1_1, ct_0, ct_1, h1, c1_w,
                          c1_b.reshape(1, 32))
    s2a, s2b = conv2(h2a, h2b, srcp, dstm, zb16)
    h3a, h3b, h3c, h3d = _comb2(s2a, s2b, h2a, h2b, rc, c2_w,
                                c2_b.reshape(1, 64))
    s3 = conv3(h3a, h3b, h3c, h3d, srcp, dstm, zb16)
    p, loss = _final(s3, (h3a, h3b, h3c, h3d), rc, labels, pos.reshape(1),
                     c3_w, c3_b.reshape(1, 128), lin2_w,
                     lin2_b.reshape(1, 1024), lin3_w, lin3_b.reshape(1, 1))
    return loss[0, 0], p
